# Initial kernel scaffold; baseline (speedup 1.0000x reference)
import jax
import jax.numpy as jnp
from jax.experimental import pallas as pl


def kernel(users, seqs, posItems, negItems, embedUserTable, embedItemTable):
    raise NotImplementedError("write your pallas kernel here")


# speedup vs baseline: 1.1934x; 1.1934x over previous
import functools

import jax
import jax.numpy as jnp
from jax import lax
from jax.experimental import pallas as pl
from jax.experimental.pallas import tpu as pltpu
from jax.experimental.pallas import tpu_sc as plsc

B = 16384
D = 128
NC = 2
NS = 16
NW = NC * NS
BPW = B // NW
CH = 128
NCHUNK = BPW // CH
L = 16
GPC = CH // L
DU = 8


def _scores_kernel(users_hbm, pos_hbm, neg_hbm, ut_hbm, it_hbm,
                   pos_out, neg_out,
                   uidx, pidx, nidx, ubuf, pbuf, nbuf, psc, nsc, sem):
    wid = lax.axis_index("s") * NC + lax.axis_index("c")
    base = wid * BPW

    for c in range(NCHUNK):
        cbase = base + c * CH
        pltpu.sync_copy(users_hbm.at[pl.ds(cbase, CH)], uidx.at[c])
        pltpu.sync_copy(pos_hbm.at[pl.ds(cbase, CH)], pidx.at[c])
        pltpu.sync_copy(neg_hbm.at[pl.ds(cbase, CH)], nidx.at[c])
        cu = pltpu.async_copy(ut_hbm.at[uidx.at[c]], ubuf, sem)
        cp = pltpu.async_copy(it_hbm.at[pidx.at[c]], pbuf, sem)
        cn = pltpu.async_copy(it_hbm.at[nidx.at[c]], nbuf, sem)
        cu.wait()
        cp.wait()
        cn.wait()

        lane_iota = lax.iota(jnp.int32, L)

        def gbody(g, carry, c=c):
            accp = jnp.zeros((L,), jnp.float32)
            accn = jnp.zeros((L,), jnp.float32)
            base_r = g * L
            for k in range(L):
                r = base_r + k
                uv = [ubuf[r, pl.ds(j * L, L)] for j in range(D // L)]
                pv = [pbuf[r, pl.ds(j * L, L)] for j in range(D // L)]
                nv = [nbuf[r, pl.ds(j * L, L)] for j in range(D // L)]
                ap = uv[0] * pv[0]
                an = uv[0] * nv[0]
                for j in range(1, D // L):
                    ap = ap + uv[j] * pv[j]
                    an = an + uv[j] * nv[j]
                m = lane_iota == k
                accp = jnp.where(m, jnp.sum(ap), accp)
                accn = jnp.where(m, jnp.sum(an), accn)
            psc[pl.ds(c * CH + base_r, L)] = accp
            nsc[pl.ds(c * CH + base_r, L)] = accn
            return carry

        lax.fori_loop(0, GPC, gbody, 0)

    pltpu.sync_copy(psc, pos_out.at[pl.ds(base, BPW)])
    pltpu.sync_copy(nsc, neg_out.at[pl.ds(base, BPW)])


@jax.jit
def _scores(users, posItems, negItems, embedUserTable, embedItemTable):
    mesh = plsc.VectorSubcoreMesh(core_axis_name="c", subcore_axis_name="s")
    run = functools.partial(
        pl.kernel,
        mesh=mesh,
        compiler_params=pltpu.CompilerParams(needs_layout_passes=False),
        out_type=(
            jax.ShapeDtypeStruct((B,), jnp.float32),
            jax.ShapeDtypeStruct((B,), jnp.float32),
        ),
        scratch_types=[
            pltpu.VMEM((NCHUNK, CH), jnp.int32),
            pltpu.VMEM((NCHUNK, CH), jnp.int32),
            pltpu.VMEM((NCHUNK, CH), jnp.int32),
            pltpu.VMEM((CH, D), jnp.float32),
            pltpu.VMEM((CH, D), jnp.float32),
            pltpu.VMEM((CH, D), jnp.float32),
            pltpu.VMEM((BPW,), jnp.float32),
            pltpu.VMEM((BPW,), jnp.float32),
            pltpu.SemaphoreType.DMA,
        ],
    )(_scores_kernel)
    return run(users, posItems, negItems, embedUserTable, embedItemTable)


def kernel(users, seqs, posItems, negItems, embedUserTable, embedItemTable):
    del seqs
    return _scores(users.astype(jnp.int32), posItems.astype(jnp.int32),
                   negItems.astype(jnp.int32), embedUserTable, embedItemTable)


# speedup vs baseline: 1.3352x; 1.1188x over previous
import functools

import jax
import jax.numpy as jnp
from jax import lax
from jax.experimental import pallas as pl
from jax.experimental.pallas import tpu as pltpu
from jax.experimental.pallas import tpu_sc as plsc

B = 16384
D = 128
NC = 2
NS = 16
NW = NC * NS
BPW = B // NW
CH = 128
NCHUNK = BPW // CH
L = 16
GPC = CH // L
DU = 8


def _scores_kernel(users_hbm, pos_hbm, neg_hbm, ut_hbm, it_hbm,
                   pos_out, neg_out,
                   uidx, pidx, nidx,
                   ubuf0, pbuf0, nbuf0, ubuf1, pbuf1, nbuf1,
                   psc, nsc, sem0, sem1):
    wid = lax.axis_index("s") * NC + lax.axis_index("c")
    base = wid * BPW
    ubufs = (ubuf0, ubuf1)
    pbufs = (pbuf0, pbuf1)
    nbufs = (nbuf0, nbuf1)
    sems = (sem0, sem1)

    for c in range(NCHUNK):
        cbase = base + c * CH
        pltpu.sync_copy(users_hbm.at[pl.ds(cbase, CH)], uidx.at[c])
        pltpu.sync_copy(pos_hbm.at[pl.ds(cbase, CH)], pidx.at[c])
        pltpu.sync_copy(neg_hbm.at[pl.ds(cbase, CH)], nidx.at[c])

    def fire(c, s):
        return (pltpu.async_copy(ut_hbm.at[uidx.at[c]], ubufs[s], sems[s]),
                pltpu.async_copy(it_hbm.at[pidx.at[c]], pbufs[s], sems[s]),
                pltpu.async_copy(it_hbm.at[nidx.at[c]], nbufs[s], sems[s]))

    inflight = {0: fire(0, 0)}
    for c in range(NCHUNK):
        s = c % 2
        if c + 1 < NCHUNK:
            inflight[c + 1] = fire(c + 1, (c + 1) % 2)
        for h in inflight.pop(c):
            h.wait()
        ubuf, pbuf, nbuf = ubufs[s], pbufs[s], nbufs[s]

        lane_iota = lax.iota(jnp.int32, L)

        def gbody(g, carry, c=c):
            accp = jnp.zeros((L,), jnp.float32)
            accn = jnp.zeros((L,), jnp.float32)
            base_r = g * L
            for k in range(L):
                r = base_r + k
                uv = [ubuf[r, pl.ds(j * L, L)] for j in range(D // L)]
                pv = [pbuf[r, pl.ds(j * L, L)] for j in range(D // L)]
                nv = [nbuf[r, pl.ds(j * L, L)] for j in range(D // L)]
                ap = uv[0] * pv[0]
                an = uv[0] * nv[0]
                for j in range(1, D // L):
                    ap = ap + uv[j] * pv[j]
                    an = an + uv[j] * nv[j]
                m = lane_iota == k
                accp = jnp.where(m, jnp.sum(ap), accp)
                accn = jnp.where(m, jnp.sum(an), accn)
            psc[pl.ds(c * CH + base_r, L)] = accp
            nsc[pl.ds(c * CH + base_r, L)] = accn
            return carry

        lax.fori_loop(0, GPC, gbody, 0)

    pltpu.sync_copy(psc, pos_out.at[pl.ds(base, BPW)])
    pltpu.sync_copy(nsc, neg_out.at[pl.ds(base, BPW)])


@jax.jit
def _scores(users, posItems, negItems, embedUserTable, embedItemTable):
    mesh = plsc.VectorSubcoreMesh(core_axis_name="c", subcore_axis_name="s")
    run = functools.partial(
        pl.kernel,
        mesh=mesh,
        compiler_params=pltpu.CompilerParams(needs_layout_passes=False),
        out_type=(
            jax.ShapeDtypeStruct((B,), jnp.float32),
            jax.ShapeDtypeStruct((B,), jnp.float32),
        ),
        scratch_types=[
            pltpu.VMEM((NCHUNK, CH), jnp.int32),
            pltpu.VMEM((NCHUNK, CH), jnp.int32),
            pltpu.VMEM((NCHUNK, CH), jnp.int32),
            pltpu.VMEM((CH, D), jnp.float32),
            pltpu.VMEM((CH, D), jnp.float32),
            pltpu.VMEM((CH, D), jnp.float32),
            pltpu.VMEM((CH, D), jnp.float32),
            pltpu.VMEM((CH, D), jnp.float32),
            pltpu.VMEM((CH, D), jnp.float32),
            pltpu.VMEM((BPW,), jnp.float32),
            pltpu.VMEM((BPW,), jnp.float32),
            pltpu.SemaphoreType.DMA,
            pltpu.SemaphoreType.DMA,
        ],
    )(_scores_kernel)
    return run(users, posItems, negItems, embedUserTable, embedItemTable)


def kernel(users, seqs, posItems, negItems, embedUserTable, embedItemTable):
    del seqs
    return _scores(users.astype(jnp.int32), posItems.astype(jnp.int32),
                   negItems.astype(jnp.int32), embedUserTable, embedItemTable)
